# baseline (speedup 1.0000x reference)
import functools

import jax
import jax.numpy as jnp
from jax import lax
from jax.experimental import pallas as pl
from jax.experimental.pallas import tpu as pltpu
from jax.experimental.pallas import tpu_sc as plsc

OUT = 7
S = 14
NBOX = 512
C = 256
N = 2
LEVEL_H = (256, 128, 64, 32)


def _pack_level(x, H):
    u = lax.bitcast_convert_type(x, jnp.uint32)

    def rne(v):
        return (v + jnp.uint32(0x7FFF) + ((v >> 16) & jnp.uint32(1))) >> 16

    word = rne(u[:, :C // 2]) | (rne(u[:, C // 2:]) << 16)
    word = lax.optimization_barrier(
        lax.bitcast_convert_type(word, jnp.int32))
    return word.transpose(0, 2, 3, 1).reshape(N * H * H, C // 2)


def _prep_body(bx_ref, idx_ref, w_ref, lvl_ref):
    f32 = jnp.float32
    x1 = bx_ref[:, 0:1]
    y1 = bx_ref[:, 1:2]
    x2 = bx_ref[:, 2:3]
    y2 = bx_ref[:, 3:4]
    area = (x2 - x1) * (y2 - y1)
    v = jnp.sqrt(area) / 224.0 + 1e-8
    lvl = ((v >= 0.5).astype(jnp.int32) + (v >= 1.0).astype(jnp.int32)
           + (v >= 2.0).astype(jnp.int32))
    scale = jnp.where(lvl == 0, f32(0.25),
             jnp.where(lvl == 1, f32(0.125),
              jnp.where(lvl == 2, f32(0.0625), f32(0.03125))))
    Hn = jnp.where(lvl == 0, 256, jnp.where(lvl == 1, 128,
          jnp.where(lvl == 2, 64, 32)))
    Hf = Hn.astype(f32)
    x1s = x1 * scale - 0.5
    y1s = y1 * scale - 0.5
    bw = (x2 * scale - 0.5 - x1s) / f32(OUT)
    bh = (y2 * scale - 0.5 - y1s) / f32(OUT)

    j = lax.broadcasted_iota(jnp.int32, (NBOX, 784), 1)
    box = lax.broadcasted_iota(jnp.int32, (NBOX, 784), 0)
    p = j // 16
    py = p // OUT
    px = p - py * OUT
    r = j - p * 16
    uy = r // 8
    ux = (r - uy * 8) // 4
    corner = r - uy * 8 - ux * 4
    sy = 2 * py + uy
    sx = 2 * px + ux
    yy = y1s + (sy.astype(f32) * 0.5 + 0.25) * bh
    xx = x1s + (sx.astype(f32) * 0.5 + 0.25) * bw
    valid = ((yy >= -1.0) & (yy <= Hf)) & ((xx >= -1.0) & (xx <= Hf))
    y = jnp.maximum(yy, 0.0)
    x = jnp.maximum(xx, 0.0)
    y0 = jnp.floor(y).astype(jnp.int32)
    x0 = jnp.floor(x).astype(jnp.int32)
    ych = y0 >= Hn - 1
    xch = x0 >= Hn - 1
    y_low = jnp.where(ych, Hn - 1, y0)
    y_high = jnp.where(ych, Hn - 1, y0 + 1)
    yc = jnp.where(ych, Hf - 1.0, y)
    x_low = jnp.where(xch, Hn - 1, x0)
    x_high = jnp.where(xch, Hn - 1, x0 + 1)
    xc = jnp.where(xch, Hf - 1.0, x)
    ly = yc - y_low.astype(f32)
    lx = xc - x_low.astype(f32)
    hy = 1.0 - ly
    hx = 1.0 - lx
    vm = jnp.where(valid, f32(0.25), f32(0.0))

    ysel = corner >= 2
    xsel = (corner - (corner // 2) * 2) == 1
    cy = jnp.where(ysel, y_high, y_low)
    cx = jnp.where(xsel, x_high, x_low)
    wy = jnp.where(ysel, ly, hy)
    wx = jnp.where(xsel, lx, hx)

    b = box // 256
    idx_ref[...] = b * (Hn * Hn) + cy * Hn + cx
    w_ref[...] = wy * wx * vm
    lvl_ref[...] = lvl


_prep = pl.pallas_call(
    _prep_body,
    out_shape=[
        jax.ShapeDtypeStruct((NBOX, 784), jnp.int32),
        jax.ShapeDtypeStruct((NBOX, 784), jnp.float32),
        jax.ShapeDtypeStruct((NBOX, 1), jnp.int32),
    ],
)


def _lane_bcast(vec, j):
    return lax.gather(
        vec,
        jnp.full((16, 1), j, jnp.int32),
        lax.GatherDimensionNumbers(
            offset_dims=(), collapsed_slice_dims=(0,), start_index_map=(0,)),
        (1,),
        mode=lax.GatherScatterMode.PROMISE_IN_BOUNDS,
    )


def _sc_pool(t0, t1, t2, t3, idxflat, wflat, lvl):
    mesh = plsc.VectorSubcoreMesh(core_axis_name="c", subcore_axis_name="s")

    @functools.partial(
        pl.kernel,
        mesh=mesh,
        out_type=jax.ShapeDtypeStruct((NBOX, 49, C), jnp.float32),
        scratch_types=[
            pltpu.VMEM((16,), jnp.int32),
            pltpu.VMEM((16 * 784,), jnp.int32),
            pltpu.VMEM((16 * 784,), jnp.float32),
            pltpu.VMEM((2, 112, C // 2), jnp.int32),
            pltpu.VMEM((2, 49, C), jnp.float32),
            pltpu.SemaphoreType.DMA,
            pltpu.SemaphoreType.DMA,
            pltpu.SemaphoreType.DMA,
            pltpu.SemaphoreType.DMA,
        ],
        compiler_params=pltpu.CompilerParams(needs_layout_passes=False),
    )
    def k(t0_hbm, t1_hbm, t2_hbm, t3_hbm, idx_hbm, w_hbm, lvl_hbm, out_hbm,
          lvl_v, idx_v, w_v, rows_v, acc_v, sem0, sem1, sem_o0, sem_o1):
        wid = lax.axis_index("s") * 2 + lax.axis_index("c")
        tabs = [t0_hbm, t1_hbm, t2_hbm, t3_hbm]
        sems = [sem0, sem1]
        osems = [sem_o0, sem_o1]
        pltpu.sync_copy(lvl_hbm.at[pl.ds(wid * 16, 16)], lvl_v)
        pltpu.sync_copy(idx_hbm.at[pl.ds(wid * (16 * 784), 16 * 784)], idx_v)
        pltpu.sync_copy(w_hbm.at[pl.ds(wid * (16 * 784), 16 * 784)], w_v)
        lvl16 = lvl_v[...]

        def box_body(t, carry):
            bi = wid * 16 + t
            par = t & 1
            lv = jnp.max(_lane_bcast(lvl16, t))

            def issue(c, nb):
                for L in range(4):
                    @pl.when(lv == L)
                    def _(L=L, c=c, nb=nb):
                        pltpu.async_copy(
                            tabs[L].at[idx_v.at[pl.ds(t * 784 + c * 112, 112)]],
                            rows_v.at[nb], sems[nb])

            for q in range(2):
                @pl.when((t >= 2) & (par == q))
                def _(q=q):
                    pltpu.make_async_copy(
                        acc_v.at[q], out_hbm.at[bi], osems[q]).wait()

            issue(0, 0)
            for c in range(7):
                b = c % 2
                if c + 1 < 7:
                    issue(c + 1, (c + 1) % 2)
                pltpu.make_async_copy(
                    tabs[0].at[idx_v.at[pl.ds(t * 784 + c * 112, 112)]],
                    rows_v.at[b], sems[b]).wait()

                def px_body(p, carry3, c=c, b=b):
                    off = t * 784 + c * 112 + p * 16
                    wvec = w_v[pl.ds(off, 16)]
                    acc = [jnp.zeros((16,), jnp.float32)
                           for _ in range(C // 16)]
                    for j in range(16):
                        wj = _lane_bcast(wvec, j)
                        r = p * 16 + j
                        for cc in range(C // 32):
                            pw = rows_v[b, r, pl.ds(cc * 16, 16)]
                            bf = plsc.bitcast(pw, jnp.bfloat16)
                            lo, hi = plsc.unpack(
                                bf, format=plsc.PackFormat.INTERLEAVED)
                            acc[cc] = acc[cc] + wj * lo
                            acc[cc + 8] = acc[cc + 8] + wj * hi
                    pg = c * 7 + p
                    for cc in range(C // 16):
                        acc_v[par, pg, pl.ds(cc * 16, 16)] = acc[cc]
                    return carry3

                lax.fori_loop(0, 7, px_body, 0)
            for q in range(2):
                @pl.when(par == q)
                def _(q=q):
                    pltpu.async_copy(acc_v.at[q], out_hbm.at[bi], osems[q])
            return carry

        lax.fori_loop(0, 16, box_body, 0)
        for q in range(2):
            pltpu.make_async_copy(
                acc_v.at[q], out_hbm.at[0], osems[q]).wait()

    return k(t0, t1, t2, t3, idxflat, wflat, lvl)


def kernel(x_p2, x_p3, x_p4, x_p5, boxes):
    tables = [_pack_level(x, H) for x, H in
              zip((x_p2, x_p3, x_p4, x_p5), LEVEL_H)]

    idx, w, lvl = _prep(boxes.reshape(NBOX, 4))
    out = _sc_pool(*tables, idx.reshape(-1), w.reshape(-1), lvl.reshape(NBOX))

    return out.transpose(0, 2, 1).reshape(NBOX, C, OUT, OUT)


# speedup vs baseline: 1.2630x; 1.2630x over previous
import functools

import jax
import jax.numpy as jnp
from jax import lax
from jax.experimental import pallas as pl
from jax.experimental.pallas import tpu as pltpu
from jax.experimental.pallas import tpu_sc as plsc

OUT = 7
S = 14
NBOX = 512
C = 256
N = 2
LEVEL_H = (256, 128, 64, 32)


def _pack_level(x, H):
    def rne(v):
        u = lax.bitcast_convert_type(v, jnp.uint32)
        return (u + jnp.uint32(0x7FFF) + ((u >> 16) & jnp.uint32(1))) >> 16

    word = rne(x[:, :C // 2]) | (rne(x[:, C // 2:]) << 16)
    word = lax.optimization_barrier(
        lax.bitcast_convert_type(word, jnp.int32))
    return word.transpose(0, 2, 3, 1).reshape(N * H * H, C // 2)


def _prep_body(bx_ref, idx_ref, w_ref, lvl_ref):
    f32 = jnp.float32
    x1 = bx_ref[:, 0:1]
    y1 = bx_ref[:, 1:2]
    x2 = bx_ref[:, 2:3]
    y2 = bx_ref[:, 3:4]
    area = (x2 - x1) * (y2 - y1)
    v = jnp.sqrt(area) / 224.0 + 1e-8
    lvl = ((v >= 0.5).astype(jnp.int32) + (v >= 1.0).astype(jnp.int32)
           + (v >= 2.0).astype(jnp.int32))
    scale = jnp.where(lvl == 0, f32(0.25),
             jnp.where(lvl == 1, f32(0.125),
              jnp.where(lvl == 2, f32(0.0625), f32(0.03125))))
    Hn = jnp.where(lvl == 0, 256, jnp.where(lvl == 1, 128,
          jnp.where(lvl == 2, 64, 32)))
    Hf = Hn.astype(f32)
    x1s = x1 * scale - 0.5
    y1s = y1 * scale - 0.5
    bw = (x2 * scale - 0.5 - x1s) / f32(OUT)
    bh = (y2 * scale - 0.5 - y1s) / f32(OUT)

    j = lax.broadcasted_iota(jnp.int32, (NBOX, 784), 1)
    box = lax.broadcasted_iota(jnp.int32, (NBOX, 784), 0)
    p = j // 16
    py = p // OUT
    px = p - py * OUT
    r = j - p * 16
    uy = r // 8
    ux = (r - uy * 8) // 4
    corner = r - uy * 8 - ux * 4
    sy = 2 * py + uy
    sx = 2 * px + ux
    yy = y1s + (sy.astype(f32) * 0.5 + 0.25) * bh
    xx = x1s + (sx.astype(f32) * 0.5 + 0.25) * bw
    valid = ((yy >= -1.0) & (yy <= Hf)) & ((xx >= -1.0) & (xx <= Hf))
    y = jnp.maximum(yy, 0.0)
    x = jnp.maximum(xx, 0.0)
    y0 = jnp.floor(y).astype(jnp.int32)
    x0 = jnp.floor(x).astype(jnp.int32)
    ych = y0 >= Hn - 1
    xch = x0 >= Hn - 1
    y_low = jnp.where(ych, Hn - 1, y0)
    y_high = jnp.where(ych, Hn - 1, y0 + 1)
    yc = jnp.where(ych, Hf - 1.0, y)
    x_low = jnp.where(xch, Hn - 1, x0)
    x_high = jnp.where(xch, Hn - 1, x0 + 1)
    xc = jnp.where(xch, Hf - 1.0, x)
    ly = yc - y_low.astype(f32)
    lx = xc - x_low.astype(f32)
    hy = 1.0 - ly
    hx = 1.0 - lx
    vm = jnp.where(valid, f32(0.25), f32(0.0))

    ysel = corner >= 2
    xsel = (corner - (corner // 2) * 2) == 1
    cy = jnp.where(ysel, y_high, y_low)
    cx = jnp.where(xsel, x_high, x_low)
    wy = jnp.where(ysel, ly, hy)
    wx = jnp.where(xsel, lx, hx)

    b = box // 256
    idx_ref[...] = b * (Hn * Hn) + cy * Hn + cx
    w_ref[...] = wy * wx * vm
    lvl_ref[...] = lvl


_prep = pl.pallas_call(
    _prep_body,
    out_shape=[
        jax.ShapeDtypeStruct((NBOX, 784), jnp.int32),
        jax.ShapeDtypeStruct((NBOX, 784), jnp.float32),
        jax.ShapeDtypeStruct((NBOX, 1), jnp.int32),
    ],
)


def _lane_bcast(vec, j):
    return lax.gather(
        vec,
        jnp.full((16, 1), j, jnp.int32),
        lax.GatherDimensionNumbers(
            offset_dims=(), collapsed_slice_dims=(0,), start_index_map=(0,)),
        (1,),
        mode=lax.GatherScatterMode.PROMISE_IN_BOUNDS,
    )


def _sc_pool(t0, t1, t2, t3, idxflat, wflat, lvl):
    mesh = plsc.VectorSubcoreMesh(core_axis_name="c", subcore_axis_name="s")

    @functools.partial(
        pl.kernel,
        mesh=mesh,
        out_type=jax.ShapeDtypeStruct((NBOX, 49, C), jnp.float32),
        scratch_types=[
            pltpu.VMEM((16,), jnp.int32),
            pltpu.VMEM((16 * 784,), jnp.int32),
            pltpu.VMEM((16 * 784,), jnp.float32),
            pltpu.VMEM((2, 112, C // 2), jnp.int32),
            pltpu.VMEM((2, 49, C), jnp.float32),
            pltpu.SemaphoreType.DMA,
            pltpu.SemaphoreType.DMA,
            pltpu.SemaphoreType.DMA,
            pltpu.SemaphoreType.DMA,
        ],
        compiler_params=pltpu.CompilerParams(needs_layout_passes=False),
    )
    def k(t0_hbm, t1_hbm, t2_hbm, t3_hbm, idx_hbm, w_hbm, lvl_hbm, out_hbm,
          lvl_v, idx_v, w_v, rows_v, acc_v, sem0, sem1, sem_o0, sem_o1):
        wid = lax.axis_index("s") * 2 + lax.axis_index("c")
        tabs = [t0_hbm, t1_hbm, t2_hbm, t3_hbm]
        sems = [sem0, sem1]
        osems = [sem_o0, sem_o1]
        pltpu.sync_copy(lvl_hbm.at[pl.ds(wid * 16, 16)], lvl_v)
        pltpu.sync_copy(idx_hbm.at[pl.ds(wid * (16 * 784), 16 * 784)], idx_v)
        pltpu.sync_copy(w_hbm.at[pl.ds(wid * (16 * 784), 16 * 784)], w_v)
        lvl16 = lvl_v[...]

        def box_body(t, carry):
            bi = wid * 16 + t
            par = t & 1
            lv = jnp.max(_lane_bcast(lvl16, t))

            def issue(c, nb):
                for L in range(4):
                    @pl.when(lv == L)
                    def _(L=L, c=c, nb=nb):
                        pltpu.async_copy(
                            tabs[L].at[idx_v.at[pl.ds(t * 784 + c * 112, 112)]],
                            rows_v.at[nb], sems[nb])

            for q in range(2):
                @pl.when((t >= 2) & (par == q))
                def _(q=q):
                    pltpu.make_async_copy(
                        acc_v.at[q], out_hbm.at[bi], osems[q]).wait()

            issue(0, 0)
            for c in range(7):
                b = c % 2
                if c + 1 < 7:
                    issue(c + 1, (c + 1) % 2)
                pltpu.make_async_copy(
                    tabs[0].at[idx_v.at[pl.ds(t * 784 + c * 112, 112)]],
                    rows_v.at[b], sems[b]).wait()

                def px_body(p, carry3, c=c, b=b):
                    off = t * 784 + c * 112 + p * 16
                    wvec = w_v[pl.ds(off, 16)]
                    acc = [jnp.zeros((16,), jnp.float32)
                           for _ in range(C // 16)]
                    for j in range(16):
                        wj = _lane_bcast(wvec, j)
                        r = p * 16 + j
                        for cc in range(C // 32):
                            pw = rows_v[b, r, pl.ds(cc * 16, 16)]
                            bf = plsc.bitcast(pw, jnp.bfloat16)
                            lo, hi = plsc.unpack(
                                bf, format=plsc.PackFormat.INTERLEAVED)
                            acc[cc] = acc[cc] + wj * lo
                            acc[cc + 8] = acc[cc + 8] + wj * hi
                    pg = c * 7 + p
                    for cc in range(C // 16):
                        acc_v[par, pg, pl.ds(cc * 16, 16)] = acc[cc]
                    return carry3

                lax.fori_loop(0, 7, px_body, 0)
            for q in range(2):
                @pl.when(par == q)
                def _(q=q):
                    pltpu.async_copy(acc_v.at[q], out_hbm.at[bi], osems[q])
            return carry

        lax.fori_loop(0, 16, box_body, 0)
        for q in range(2):
            pltpu.make_async_copy(
                acc_v.at[q], out_hbm.at[0], osems[q]).wait()

    return k(t0, t1, t2, t3, idxflat, wflat, lvl)


def kernel(x_p2, x_p3, x_p4, x_p5, boxes):
    tables = [_pack_level(x, H) for x, H in
              zip((x_p2, x_p3, x_p4, x_p5), LEVEL_H)]

    idx, w, lvl = _prep(boxes.reshape(NBOX, 4))
    out = _sc_pool(*tables, idx.reshape(-1), w.reshape(-1), lvl.reshape(NBOX))

    return out.transpose(0, 2, 1).reshape(NBOX, C, OUT, OUT)
